# baseline (speedup 1.0000x reference)
import functools

import jax
import jax.numpy as jnp
from jax import lax
from jax.experimental import pallas as pl
from jax.experimental.pallas import tpu as pltpu
from jax.experimental.pallas import tpu_sc as plsc

_TN = 1024
_GROUPS = 8
_EPS = 1e-5
_GW = 128
_TT = 1024


def _nn_body(ptsT_ref, cT_ref, feats_ref, temb_ref, W_ref, b_ref, g_ref, bt_ref,
             idx_ref, zfin_ref, z_ref, cnt_ref):
    M = cT_ref.shape[1]
    OC = W_ref.shape[0]
    NT = idx_ref.shape[1]
    N = NT * _TN
    b = pl.program_id(0)
    j = pl.program_id(1)

    @pl.when(j == 0)
    def _init():
        z_ref[...] = jnp.dot(W_ref[...], feats_ref[0],
                             preferred_element_type=jnp.float32) + b_ref[...]
        cnt_ref[...] = jnp.zeros_like(cnt_ref)

    @pl.when(j < NT)
    def _tile():
        p = ptsT_ref[0]
        cT = cT_ref[0]
        pn2 = jnp.sum(p * p, axis=1, keepdims=True)
        cn2 = jnp.sum(cT * cT, axis=1).reshape(1, M)
        e = lax.dot_general(p, cT, (((1,), (1,)), ((), ())),
                            preferred_element_type=jnp.float32)
        score = (pn2 + cn2) - 2.0 * e
        idx = jnp.argmin(score, axis=1).reshape(1, _TN)
        idx_ref[0, pl.ds(j, 1), :] = idx + b * M
        iota_m = lax.broadcasted_iota(jnp.int32, (_TN, M), 1)
        onehot = (iota_m == idx.reshape(_TN, 1)).astype(jnp.float32)
        cnt_ref[...] += jnp.sum(onehot, axis=0, keepdims=True)

    @pl.when(j == NT)
    def _epilogue():
        cpg = OC // _GROUPS
        G = (lax.broadcasted_iota(jnp.int32, (_GROUPS, OC), 1) // cpg
             == lax.broadcasted_iota(jnp.int32, (_GROUPS, OC), 0)).astype(jnp.float32)
        GT = (lax.broadcasted_iota(jnp.int32, (OC, _GROUPS), 0) // cpg
              == lax.broadcasted_iota(jnp.int32, (OC, _GROUPS), 1)).astype(jnp.float32)
        z = z_ref[...]
        cnt = cnt_ref[...]
        s1 = jnp.sum(z * cnt, axis=1, keepdims=True)
        s2 = jnp.sum(z * z * cnt, axis=1, keepdims=True)
        denom = float(N * cpg)
        gmean = jnp.dot(G, s1, preferred_element_type=jnp.float32) / denom
        gm2 = jnp.dot(G, s2, preferred_element_type=jnp.float32) / denom
        gscale = lax.rsqrt(gm2 - gmean * gmean + _EPS)
        mean_c = jnp.dot(GT, gmean, preferred_element_type=jnp.float32)
        scale_c = jnp.dot(GT, gscale, preferred_element_type=jnp.float32)
        a = g_ref[...] * scale_c
        c0 = bt_ref[...] - g_ref[...] * scale_c * mean_c
        yn = z * a + c0
        zfin_ref[0] = yn * jax.nn.sigmoid(yn)


def _nn_and_table(ptsT, cT, feats, temb, W, b2, g2, bt2):
    B, N, _ = ptsT.shape
    M = cT.shape[1]
    C = feats.shape[1]
    OC = W.shape[0]
    NT = N // _TN
    last = NT - 1
    idx, zfin = pl.pallas_call(
        _nn_body,
        grid=(B, NT + 1),
        in_specs=[
            pl.BlockSpec((1, _TN, 3), lambda i, j: (i, jnp.minimum(j, last), 0)),
            pl.BlockSpec((1, M, 3), lambda i, j: (i, 0, 0)),
            pl.BlockSpec((1, C, M), lambda i, j: (i, 0, 0)),
            pl.BlockSpec((1, C, M), lambda i, j: (i, 0, 0)),
            pl.BlockSpec((OC, C), lambda i, j: (0, 0)),
            pl.BlockSpec((OC, 1), lambda i, j: (0, 0)),
            pl.BlockSpec((OC, 1), lambda i, j: (0, 0)),
            pl.BlockSpec((OC, 1), lambda i, j: (0, 0)),
        ],
        out_specs=[
            pl.BlockSpec((1, NT, _TN), lambda i, j: (i, 0, 0)),
            pl.BlockSpec((1, OC, M), lambda i, j: (i, 0, 0)),
        ],
        out_shape=[
            jax.ShapeDtypeStruct((B, NT, _TN), jnp.int32),
            jax.ShapeDtypeStruct((B, OC, M), jnp.float32),
        ],
        scratch_shapes=[
            pltpu.VMEM((OC, M), jnp.float32),
            pltpu.VMEM((1, M), jnp.float32),
        ],
    )(ptsT, cT, feats, temb, W, b2, g2, bt2)
    return idx, zfin


def _sc_gather(tbl, idx_flat):
    BN = idx_flat.shape[0]
    D = tbl.shape[1]
    idx2 = idx_flat.reshape(1, BN)
    mesh = plsc.VectorSubcoreMesh(core_axis_name="core", subcore_axis_name="subcore")

    @functools.partial(
        pl.kernel,
        out_type=jax.ShapeDtypeStruct((BN, D), jnp.float32),
        mesh=mesh,
    )
    def k(tbl_hbm, i_hbm, o_hbm):
        def body(i_vmem, o_vmem):
            pltpu.sync_copy(tbl_hbm.at[i_vmem.at[0]], o_vmem)

        pltpu.emit_pipeline(
            body,
            grid=(BN // _GW,),
            in_specs=[pl.BlockSpec((1, _GW), index_map=lambda i: (0, i))],
            out_specs=[pl.BlockSpec((_GW, D), index_map=lambda i: (i, 0))],
            core_axis_name=("core", "subcore"),
            dimension_semantics=(pltpu.PARALLEL,),
        )(i_hbm, o_hbm)

    return k(tbl, idx2)


def _tr_body(rows_ref, out_ref, tout_ref):
    OC = out_ref.shape[1]
    r = rows_ref[0]
    t = lax.transpose(r, (1, 0))
    out_ref[0] = t[0:OC]
    tout_ref[0] = t[OC:]


def _split_transpose(rows, B, N, OC):
    NT2 = N // _TT
    rows3 = rows.reshape(B, N, 2 * OC)
    return pl.pallas_call(
        _tr_body,
        grid=(B, NT2),
        in_specs=[pl.BlockSpec((1, _TT, 2 * OC), lambda i, j: (i, j, 0))],
        out_specs=[
            pl.BlockSpec((1, OC, _TT), lambda i, j: (i, 0, j)),
            pl.BlockSpec((1, OC, _TT), lambda i, j: (i, 0, j)),
        ],
        out_shape=[
            jax.ShapeDtypeStruct((B, OC, N), jnp.float32),
            jax.ShapeDtypeStruct((B, OC, N), jnp.float32),
        ],
    )(rows3)


def kernel(points_coords, centers_coords, centers_features, temb, W, b, gamma, beta):
    B, _, N = points_coords.shape
    M = centers_coords.shape[2]
    OC = W.shape[0]

    ptsT = jnp.transpose(points_coords, (0, 2, 1))
    cT = jnp.transpose(centers_coords, (0, 2, 1))
    b2 = b.reshape(OC, 1)
    g2 = gamma.reshape(OC, 1)
    bt2 = beta.reshape(OC, 1)

    outs, touts = [], []
    for i in range(B):
        idx, zfin = _nn_and_table(
            ptsT[i:i + 1], cT[i:i + 1], centers_features[i:i + 1],
            temb[i:i + 1], W, b2, g2, bt2)
        tbl = jnp.concatenate([zfin, temb[i:i + 1]], axis=1)
        tbl = jnp.transpose(tbl, (0, 2, 1)).reshape(M, 2 * OC)
        rows = _sc_gather(tbl, idx.reshape(N))
        o, t = _split_transpose(rows, 1, N, OC)
        outs.append(o)
        touts.append(t)
    out = jnp.concatenate(outs, axis=0)
    tout = jnp.concatenate(touts, axis=0)
    return (out, points_coords, tout)


# speedup vs baseline: 2.1229x; 2.1229x over previous
import jax
import jax.numpy as jnp
from jax import lax
from jax.experimental import pallas as pl
from jax.experimental.pallas import tpu as pltpu

_TN = 2048
_GROUPS = 8
_EPS = 1e-5


def _fp_body(ptsT_ref, cT_ref, feats_ref, temb_ref, W_ref, b_ref, g_ref, bt_ref,
             out_ref, tout_ref, thi_ref, tlo_ref, ufeat_ref, s1_ref, s2_ref):
    M = cT_ref.shape[1]
    N = ufeat_ref.shape[1]
    OC = W_ref.shape[0]
    NT = N // _TN
    j = pl.program_id(1)

    @pl.when(j == 0)
    def _init():
        z = jnp.dot(W_ref[...], feats_ref[0],
                    preferred_element_type=jnp.float32) + b_ref[...]
        table = jnp.concatenate([z, temb_ref[0]], axis=0)
        thi = table.astype(jnp.bfloat16)
        thi_ref[...] = thi
        tlo_ref[...] = (table - thi.astype(jnp.float32)).astype(jnp.bfloat16)
        s1_ref[...] = jnp.zeros_like(s1_ref)
        s2_ref[...] = jnp.zeros_like(s2_ref)

    @pl.when(j < NT)
    def _tile():
        p = ptsT_ref[0]
        cT = cT_ref[0]
        pn2 = jnp.sum(p * p, axis=1, keepdims=True)
        cn2 = jnp.sum(cT * cT, axis=1).reshape(1, M)
        e = lax.dot_general(p, cT, (((1,), (1,)), ((), ())),
                            preferred_element_type=jnp.float32)
        score = (pn2 + cn2) - 2.0 * e
        idx = jnp.argmin(score, axis=1).reshape(_TN, 1)
        iota_m = lax.broadcasted_iota(jnp.int32, (_TN, M), 1)
        onehot = (iota_m == idx).astype(jnp.bfloat16)
        dn = (((1,), (1,)), ((), ()))
        u = lax.dot_general(thi_ref[...], onehot, dn,
                            preferred_element_type=jnp.float32)
        uf = u[0:OC]
        st = pl.multiple_of(j * _TN, _TN)
        ufeat_ref[:, pl.ds(st, _TN)] = uf
        tout_ref[0] = u[OC:]
        s1_ref[...] += jnp.sum(uf, axis=1, keepdims=True)
        s2_ref[...] += jnp.sum(uf * uf, axis=1, keepdims=True)

    @pl.when(j == NT)
    def _epilogue():
        cpg = OC // _GROUPS
        G = (lax.broadcasted_iota(jnp.int32, (_GROUPS, OC), 1) // cpg
             == lax.broadcasted_iota(jnp.int32, (_GROUPS, OC), 0)).astype(jnp.float32)
        GT = (lax.broadcasted_iota(jnp.int32, (OC, _GROUPS), 0) // cpg
              == lax.broadcasted_iota(jnp.int32, (OC, _GROUPS), 1)).astype(jnp.float32)
        denom = float(N * cpg)
        gmean = jnp.dot(G, s1_ref[...], preferred_element_type=jnp.float32) / denom
        gm2 = jnp.dot(G, s2_ref[...], preferred_element_type=jnp.float32) / denom
        gscale = lax.rsqrt(gm2 - gmean * gmean + _EPS)
        mean_c = jnp.dot(GT, gmean, preferred_element_type=jnp.float32)
        scale_c = jnp.dot(GT, gscale, preferred_element_type=jnp.float32)
        a = g_ref[...] * scale_c
        c0 = bt_ref[...] - g_ref[...] * scale_c * mean_c
        yn = ufeat_ref[...] * a + c0
        out_ref[0] = yn * jax.nn.sigmoid(yn)


def kernel(points_coords, centers_coords, centers_features, temb, W, b, gamma, beta):
    B, _, N = points_coords.shape
    M = centers_coords.shape[2]
    C = centers_features.shape[1]
    OC = W.shape[0]
    NT = N // _TN

    ptsT = jnp.transpose(points_coords, (0, 2, 1))
    cT = jnp.transpose(centers_coords, (0, 2, 1))
    b2 = b.reshape(OC, 1)
    g2 = gamma.reshape(OC, 1)
    bt2 = beta.reshape(OC, 1)

    last = NT - 1
    out, tout = pl.pallas_call(
        _fp_body,
        grid=(B, NT + 1),
        in_specs=[
            pl.BlockSpec((1, _TN, 3), lambda i, j: (i, jnp.minimum(j, last), 0)),
            pl.BlockSpec((1, M, 3), lambda i, j: (i, 0, 0)),
            pl.BlockSpec((1, C, M), lambda i, j: (i, 0, 0)),
            pl.BlockSpec((1, C, M), lambda i, j: (i, 0, 0)),
            pl.BlockSpec((OC, C), lambda i, j: (0, 0)),
            pl.BlockSpec((OC, 1), lambda i, j: (0, 0)),
            pl.BlockSpec((OC, 1), lambda i, j: (0, 0)),
            pl.BlockSpec((OC, 1), lambda i, j: (0, 0)),
        ],
        out_specs=[
            pl.BlockSpec((1, OC, N), lambda i, j: (i, 0, 0)),
            pl.BlockSpec((1, C, _TN), lambda i, j: (i, 0, jnp.minimum(j, last))),
        ],
        out_shape=[
            jax.ShapeDtypeStruct((B, OC, N), jnp.float32),
            jax.ShapeDtypeStruct((B, C, N), jnp.float32),
        ],
        scratch_shapes=[
            pltpu.VMEM((OC + C, M), jnp.bfloat16),
            pltpu.VMEM((OC + C, M), jnp.bfloat16),
            pltpu.VMEM((OC, N), jnp.float32),
            pltpu.VMEM((OC, 1), jnp.float32),
            pltpu.VMEM((OC, 1), jnp.float32),
        ],
    )(ptsT, cT, centers_features, temb, W, b2, g2, bt2)

    return (out, points_coords, tout)
